# baseline (speedup 1.0000x reference)
import functools

import jax
import jax.numpy as jnp
from jax import lax
from jax.experimental import pallas as pl
from jax.experimental.pallas import tpu as pltpu
from jax.experimental.pallas import tpu_sc as plsc

MARGIN = 0.3
K_POS = 8
K_NEG = 64
EPS = 1e-8

LANES = 128
BLOCK = 8192
N_ROWS = 65553
SC_WORKERS = 32
SC_ROWS_PER_W = 1024
SC_ROWS = SC_WORKERS * SC_ROWS_PER_W
TC_START_BLK = SC_ROWS // BLOCK
CHUNK = 64


def _tc_cos_kernel(img_ref, anchor_ref, cos_ref):
    x = img_ref[...]
    a = anchor_ref[0:1, :]
    xb = x.astype(jnp.bfloat16)
    ab = jnp.transpose(a.astype(jnp.bfloat16))
    dot = jax.lax.dot_general(
        xb, ab, (((1,), (0,)), ((), ())),
        preferred_element_type=jnp.float32)
    ones = jnp.ones((a.shape[1], 1), jnp.bfloat16)
    sq = jax.lax.dot_general(
        xb * xb, ones, (((1,), (0,)), ((), ())),
        preferred_element_type=jnp.float32)
    na = jnp.sqrt(jnp.sum(a * a))
    rows = BLOCK // LANES
    dot_r = dot.reshape(rows, LANES)
    sq_r = sq.reshape(rows, LANES)
    denom = jnp.maximum(na * jnp.sqrt(sq_r), EPS)
    cos_ref[...] = dot_r / denom


def _sc_matvec_kernel(img_hbm, dot_hbm, sq_hbm, a_buf, bufs, acc_d, acc_s,
                      sem0, sem1):
    w = lax.axis_index("s") * 2 + lax.axis_index("c")
    base_row = w * SC_ROWS_PER_W

    pltpu.sync_copy(img_hbm.at[pl.ds(0, 1), :], a_buf)
    arow = a_buf.at[0]

    nchunks = SC_ROWS_PER_W // CHUNK
    sems = [sem0, sem1]

    def start(c):
        return pltpu.async_copy(
            img_hbm.at[pl.ds(base_row + c * CHUNK, CHUNK), :],
            bufs.at[c % 2], sems[c % 2])

    pending = {0: start(0)}
    for c in range(nchunks):
        slot = c % 2
        if c + 1 < nchunks:
            pending[c + 1] = start(c + 1)
        pending.pop(c).wait()
        buf_slot = bufs.at[slot]

        def rowbody(r, _, buf_slot=buf_slot, c=c):
            d0 = jnp.zeros((16,), jnp.float32)
            d1 = jnp.zeros((16,), jnp.float32)
            s0 = jnp.zeros((16,), jnp.float32)
            s1 = jnp.zeros((16,), jnp.float32)
            for k in range(0, 16, 2):
                xv0 = buf_slot[r, pl.ds(k * 16, 16)]
                av0 = arow[pl.ds(k * 16, 16)]
                xv1 = buf_slot[r, pl.ds((k + 1) * 16, 16)]
                av1 = arow[pl.ds((k + 1) * 16, 16)]
                d0 = d0 + xv0 * av0
                d1 = d1 + xv1 * av1
                s0 = s0 + xv0 * xv0
                s1 = s1 + xv1 * xv1
            off = (c * CHUNK + r) * 16
            acc_d[pl.ds(off, 16)] = d0 + d1
            acc_s[pl.ds(off, 16)] = s0 + s1
            return 0

        lax.fori_loop(0, CHUNK, rowbody, 0)

    np_w = SC_ROWS_PER_W * 16
    pltpu.sync_copy(acc_d, dot_hbm.at[pl.ds(w * np_w, np_w)])
    pltpu.sync_copy(acc_s, sq_hbm.at[pl.ds(w * np_w, np_w)])


def _sc_matvec(img):
    mesh = plsc.VectorSubcoreMesh(core_axis_name="c", subcore_axis_name="s")
    f = pl.kernel(
        _sc_matvec_kernel,
        mesh=mesh,
        out_type=[
            jax.ShapeDtypeStruct((SC_ROWS * 16,), jnp.float32),
            jax.ShapeDtypeStruct((SC_ROWS * 16,), jnp.float32),
        ],
        scratch_types=[
            pltpu.VMEM((1, 256), jnp.float32),
            pltpu.VMEM((2, CHUNK, 256), jnp.float32),
            pltpu.VMEM((SC_ROWS_PER_W * 16,), jnp.float32),
            pltpu.VMEM((SC_ROWS_PER_W * 16,), jnp.float32),
            pltpu.SemaphoreType.DMA,
            pltpu.SemaphoreType.DMA,
        ],
    )
    return f(img)


def _float_keys(vals, valid):
    bits = jax.lax.bitcast_convert_type(vals, jnp.uint32)
    neg = bits >= jnp.uint32(0x80000000)
    key = jnp.where(neg, ~bits, bits | jnp.uint32(0x80000000))
    return jnp.where(valid, key, jnp.uint32(0))


def _key_to_float(key):
    neg = key < jnp.uint32(0x80000000)
    bits = jnp.where(neg, ~key, key ^ jnp.uint32(0x80000000))
    return jax.lax.bitcast_convert_type(bits, jnp.float32)


def _kth_largest_2(keys_a, keys_b, k):
    prefix = jnp.uint32(0)
    for b in range(30, -1, -2):
        c3 = prefix | jnp.uint32(3 << b)
        c2 = prefix | jnp.uint32(2 << b)
        c1 = prefix | jnp.uint32(1 << b)
        n3 = (jnp.sum((keys_a >= c3).astype(jnp.int32))
              + jnp.sum((keys_b >= c3).astype(jnp.int32)))
        n2 = (jnp.sum((keys_a >= c2).astype(jnp.int32))
              + jnp.sum((keys_b >= c2).astype(jnp.int32)))
        n1 = (jnp.sum((keys_a >= c1).astype(jnp.int32))
              + jnp.sum((keys_b >= c1).astype(jnp.int32)))
        prefix = jnp.where(
            n3 >= k, c3, jnp.where(n2 >= k, c2, jnp.where(n1 >= k, c1, prefix)))
    return _key_to_float(prefix)


def _sel_kernel(cos_tc_ref, dot_sc_ref, sq_sc_ref, anchor_ref, out_ref):
    cos_tc = cos_tc_ref[...]
    a = anchor_ref[0:1, :]
    na = jnp.sqrt(jnp.sum(a * a))

    dp = dot_sc_ref[...]
    sp = sq_sc_ref[...]
    jj = jax.lax.broadcasted_iota(jnp.int32, (LANES, LANES), 0)
    gg = jax.lax.broadcasted_iota(jnp.int32, (LANES, LANES), 1)
    gmat = (jj // 16 == gg // 16).astype(jnp.float32)
    dot_b = jax.lax.dot_general(
        dp, gmat, (((1,), (0,)), ((), ())),
        preferred_element_type=jnp.float32)
    sq_b = jax.lax.dot_general(
        sp, gmat, (((1,), (0,)), ((), ())),
        preferred_element_type=jnp.float32)
    cos_b = dot_b / jnp.maximum(na * jnp.sqrt(sq_b), EPS)

    rb = jax.lax.broadcasted_iota(jnp.int32, cos_b.shape, 0)
    cb = jax.lax.broadcasted_iota(jnp.int32, cos_b.shape, 1)
    gpos = rb * 8 + cb // 16
    once = cb % 16 == 0
    valid2 = once & (gpos >= 17)

    r1 = jax.lax.broadcasted_iota(jnp.int32, cos_tc.shape, 0)
    c1 = jax.lax.broadcasted_iota(jnp.int32, cos_tc.shape, 1)
    pos1 = SC_ROWS + r1 * LANES + c1
    valid1 = pos1 < N_ROWS

    pcos = cos_b[0:8, :]
    pg = gpos[0:8, :]
    validp = (once[0:8, :]) & (pg >= 1) & (pg <= 16)
    d_p = 1.0 - pcos
    keys_p = _float_keys(d_p, validp)
    zerok = jnp.zeros((8, LANES), jnp.uint32)
    t8 = _kth_largest_2(keys_p, zerok, K_POS)
    gt8 = validp & (d_p > t8)
    g8 = jnp.sum(gt8.astype(jnp.int32))
    s8 = jnp.sum(jnp.where(gt8, d_p, 0.0))
    mean_p = (s8 + (K_POS - g8).astype(jnp.float32) * t8) / K_POS

    keys1 = _float_keys(cos_tc, valid1)
    keys2 = _float_keys(cos_b, valid2)
    t64 = _kth_largest_2(keys1, keys2, K_NEG)
    c = mean_p + MARGIN
    h1 = jnp.maximum(c - (1.0 - cos_tc), 0.0)
    h2 = jnp.maximum(c - (1.0 - cos_b), 0.0)
    gt1 = valid1 & (cos_tc > t64)
    gt2 = valid2 & (cos_b > t64)
    gn = jnp.sum(gt1.astype(jnp.int32)) + jnp.sum(gt2.astype(jnp.int32))
    sh = jnp.sum(jnp.where(gt1, h1, 0.0)) + jnp.sum(jnp.where(gt2, h2, 0.0))
    ht = jnp.maximum(c - (1.0 - t64), 0.0)
    loss = (sh + (K_NEG - gn).astype(jnp.float32) * ht) / K_NEG
    out_ref[...] = jnp.reshape(loss, (1, 1))


def kernel(img):
    n, d = img.shape
    nblocks = pl.cdiv(n - SC_ROWS, BLOCK)
    tc_rows = nblocks * BLOCK // LANES

    cos_tc = pl.pallas_call(
        _tc_cos_kernel,
        grid=(nblocks,),
        in_specs=[
            pl.BlockSpec((BLOCK, d), lambda i: (i + TC_START_BLK, 0)),
            pl.BlockSpec((8, d), lambda i: (0, 0)),
        ],
        out_specs=pl.BlockSpec((BLOCK // LANES, LANES), lambda i: (i, 0)),
        out_shape=jax.ShapeDtypeStruct((tc_rows, LANES), jnp.float32),
    )(img, img)

    dot_sc, sq_sc = _sc_matvec(img)

    out = pl.pallas_call(
        _sel_kernel,
        grid=(1,),
        in_specs=[
            pl.BlockSpec((tc_rows, LANES), lambda i: (0, 0)),
            pl.BlockSpec((SC_ROWS * 16 // LANES, LANES), lambda i: (0, 0)),
            pl.BlockSpec((SC_ROWS * 16 // LANES, LANES), lambda i: (0, 0)),
            pl.BlockSpec((8, d), lambda i: (0, 0)),
        ],
        out_specs=pl.BlockSpec((1, 1), lambda i: (0, 0)),
        out_shape=jax.ShapeDtypeStruct((1, 1), jnp.float32),
    )(cos_tc, dot_sc.reshape(SC_ROWS * 16 // LANES, LANES),
      sq_sc.reshape(SC_ROWS * 16 // LANES, LANES), img)
    return out[0, 0]


# speedup vs baseline: 1.4622x; 1.4622x over previous
import functools

import jax
import jax.numpy as jnp
from jax import lax
from jax.experimental import pallas as pl
from jax.experimental.pallas import tpu as pltpu
from jax.experimental.pallas import tpu_sc as plsc

MARGIN = 0.3
K_POS = 8
K_NEG = 64
EPS = 1e-8

LANES = 128
BLOCK = 8192
N_ROWS = 65553
SC_WORKERS = 32
SC_ROWS_PER_W = 256
SC_ROWS = SC_WORKERS * SC_ROWS_PER_W
TC_START_BLK = SC_ROWS // BLOCK
CHUNK = 64


def _tc_cos_kernel(img_ref, anchor_ref, cos_ref):
    x = img_ref[...]
    a = anchor_ref[0:1, :]
    xb = x.astype(jnp.bfloat16)
    ab = jnp.transpose(a.astype(jnp.bfloat16))
    dot = jax.lax.dot_general(
        xb, ab, (((1,), (0,)), ((), ())),
        preferred_element_type=jnp.float32)
    ones = jnp.ones((a.shape[1], 1), jnp.bfloat16)
    sq = jax.lax.dot_general(
        xb * xb, ones, (((1,), (0,)), ((), ())),
        preferred_element_type=jnp.float32)
    na = jnp.sqrt(jnp.sum(a * a))
    rows = BLOCK // LANES
    dot_r = dot.reshape(rows, LANES)
    sq_r = sq.reshape(rows, LANES)
    denom = jnp.maximum(na * jnp.sqrt(sq_r), EPS)
    cos_ref[...] = dot_r / denom


def _sc_matvec_kernel(img_hbm, dot_hbm, sq_hbm, a_buf, bufs, acc_d, acc_s,
                      sem0, sem1):
    w = lax.axis_index("s") * 2 + lax.axis_index("c")
    base_row = w * SC_ROWS_PER_W

    pltpu.sync_copy(img_hbm.at[pl.ds(0, 1), :], a_buf)
    arow = a_buf.at[0]

    nchunks = SC_ROWS_PER_W // CHUNK
    sems = [sem0, sem1]

    def start(c):
        return pltpu.async_copy(
            img_hbm.at[pl.ds(base_row + c * CHUNK, CHUNK), :],
            bufs.at[c % 2], sems[c % 2])

    pending = {0: start(0)}
    for c in range(nchunks):
        slot = c % 2
        if c + 1 < nchunks:
            pending[c + 1] = start(c + 1)
        pending.pop(c).wait()
        buf_slot = bufs.at[slot]

        def rowbody(r, _, buf_slot=buf_slot, c=c):
            d0 = jnp.zeros((16,), jnp.float32)
            d1 = jnp.zeros((16,), jnp.float32)
            s0 = jnp.zeros((16,), jnp.float32)
            s1 = jnp.zeros((16,), jnp.float32)
            for k in range(0, 16, 2):
                xv0 = buf_slot[r, pl.ds(k * 16, 16)]
                av0 = arow[pl.ds(k * 16, 16)]
                xv1 = buf_slot[r, pl.ds((k + 1) * 16, 16)]
                av1 = arow[pl.ds((k + 1) * 16, 16)]
                d0 = d0 + xv0 * av0
                d1 = d1 + xv1 * av1
                s0 = s0 + xv0 * xv0
                s1 = s1 + xv1 * xv1
            off = (c * CHUNK + r) * 16
            acc_d[pl.ds(off, 16)] = d0 + d1
            acc_s[pl.ds(off, 16)] = s0 + s1
            return 0

        lax.fori_loop(0, CHUNK, rowbody, 0)

    np_w = SC_ROWS_PER_W * 16
    pltpu.sync_copy(acc_d, dot_hbm.at[pl.ds(w * np_w, np_w)])
    pltpu.sync_copy(acc_s, sq_hbm.at[pl.ds(w * np_w, np_w)])


def _sc_matvec(img):
    mesh = plsc.VectorSubcoreMesh(core_axis_name="c", subcore_axis_name="s")
    f = pl.kernel(
        _sc_matvec_kernel,
        mesh=mesh,
        out_type=[
            jax.ShapeDtypeStruct((SC_ROWS * 16,), jnp.float32),
            jax.ShapeDtypeStruct((SC_ROWS * 16,), jnp.float32),
        ],
        scratch_types=[
            pltpu.VMEM((1, 256), jnp.float32),
            pltpu.VMEM((2, CHUNK, 256), jnp.float32),
            pltpu.VMEM((SC_ROWS_PER_W * 16,), jnp.float32),
            pltpu.VMEM((SC_ROWS_PER_W * 16,), jnp.float32),
            pltpu.SemaphoreType.DMA,
            pltpu.SemaphoreType.DMA,
        ],
    )
    return f(img)


def _float_keys(vals, valid):
    bits = jax.lax.bitcast_convert_type(vals, jnp.uint32)
    neg = bits >= jnp.uint32(0x80000000)
    key = jnp.where(neg, ~bits, bits | jnp.uint32(0x80000000))
    return jnp.where(valid, key, jnp.uint32(0))


def _key_to_float(key):
    neg = key < jnp.uint32(0x80000000)
    bits = jnp.where(neg, ~key, key ^ jnp.uint32(0x80000000))
    return jax.lax.bitcast_convert_type(bits, jnp.float32)


def _kth_largest_2(keys_a, keys_b, k):
    prefix = jnp.uint32(0)
    for b in range(30, -1, -2):
        c3 = prefix | jnp.uint32(3 << b)
        c2 = prefix | jnp.uint32(2 << b)
        c1 = prefix | jnp.uint32(1 << b)
        n3 = (jnp.sum((keys_a >= c3).astype(jnp.int32))
              + jnp.sum((keys_b >= c3).astype(jnp.int32)))
        n2 = (jnp.sum((keys_a >= c2).astype(jnp.int32))
              + jnp.sum((keys_b >= c2).astype(jnp.int32)))
        n1 = (jnp.sum((keys_a >= c1).astype(jnp.int32))
              + jnp.sum((keys_b >= c1).astype(jnp.int32)))
        prefix = jnp.where(
            n3 >= k, c3, jnp.where(n2 >= k, c2, jnp.where(n1 >= k, c1, prefix)))
    return _key_to_float(prefix)


def _sel_kernel(cos_tc_ref, dot_sc_ref, sq_sc_ref, anchor_ref, out_ref):
    cos_tc = cos_tc_ref[...]
    a = anchor_ref[0:1, :]
    na = jnp.sqrt(jnp.sum(a * a))

    dp = dot_sc_ref[...]
    sp = sq_sc_ref[...]
    jj = jax.lax.broadcasted_iota(jnp.int32, (LANES, LANES), 0)
    gg = jax.lax.broadcasted_iota(jnp.int32, (LANES, LANES), 1)
    gmat = (jj // 16 == gg // 16).astype(jnp.float32)
    dot_b = jax.lax.dot_general(
        dp, gmat, (((1,), (0,)), ((), ())),
        preferred_element_type=jnp.float32)
    sq_b = jax.lax.dot_general(
        sp, gmat, (((1,), (0,)), ((), ())),
        preferred_element_type=jnp.float32)
    cos_b = dot_b / jnp.maximum(na * jnp.sqrt(sq_b), EPS)

    rb = jax.lax.broadcasted_iota(jnp.int32, cos_b.shape, 0)
    cb = jax.lax.broadcasted_iota(jnp.int32, cos_b.shape, 1)
    gpos = rb * 8 + cb // 16
    once = cb % 16 == 0
    valid2 = once & (gpos >= 17)

    r1 = jax.lax.broadcasted_iota(jnp.int32, cos_tc.shape, 0)
    c1 = jax.lax.broadcasted_iota(jnp.int32, cos_tc.shape, 1)
    pos1 = SC_ROWS + r1 * LANES + c1
    valid1 = pos1 < N_ROWS

    pcos = cos_b[0:8, :]
    pg = gpos[0:8, :]
    validp = (once[0:8, :]) & (pg >= 1) & (pg <= 16)
    d_p = 1.0 - pcos
    keys_p = _float_keys(d_p, validp)
    zerok = jnp.zeros((8, LANES), jnp.uint32)
    t8 = _kth_largest_2(keys_p, zerok, K_POS)
    gt8 = validp & (d_p > t8)
    g8 = jnp.sum(gt8.astype(jnp.int32))
    s8 = jnp.sum(jnp.where(gt8, d_p, 0.0))
    mean_p = (s8 + (K_POS - g8).astype(jnp.float32) * t8) / K_POS

    keys1 = _float_keys(cos_tc, valid1)
    keys2 = _float_keys(cos_b, valid2)
    t64 = _kth_largest_2(keys1, keys2, K_NEG)
    c = mean_p + MARGIN
    h1 = jnp.maximum(c - (1.0 - cos_tc), 0.0)
    h2 = jnp.maximum(c - (1.0 - cos_b), 0.0)
    gt1 = valid1 & (cos_tc > t64)
    gt2 = valid2 & (cos_b > t64)
    gn = jnp.sum(gt1.astype(jnp.int32)) + jnp.sum(gt2.astype(jnp.int32))
    sh = jnp.sum(jnp.where(gt1, h1, 0.0)) + jnp.sum(jnp.where(gt2, h2, 0.0))
    ht = jnp.maximum(c - (1.0 - t64), 0.0)
    loss = (sh + (K_NEG - gn).astype(jnp.float32) * ht) / K_NEG
    out_ref[...] = jnp.reshape(loss, (1, 1))


def kernel(img):
    n, d = img.shape
    nblocks = pl.cdiv(n - SC_ROWS, BLOCK)
    tc_rows = nblocks * BLOCK // LANES

    cos_tc = pl.pallas_call(
        _tc_cos_kernel,
        grid=(nblocks,),
        in_specs=[
            pl.BlockSpec((BLOCK, d), lambda i: (i + TC_START_BLK, 0)),
            pl.BlockSpec((8, d), lambda i: (0, 0)),
        ],
        out_specs=pl.BlockSpec((BLOCK // LANES, LANES), lambda i: (i, 0)),
        out_shape=jax.ShapeDtypeStruct((tc_rows, LANES), jnp.float32),
    )(img, img)

    dot_sc, sq_sc = _sc_matvec(img)

    out = pl.pallas_call(
        _sel_kernel,
        grid=(1,),
        in_specs=[
            pl.BlockSpec((tc_rows, LANES), lambda i: (0, 0)),
            pl.BlockSpec((SC_ROWS * 16 // LANES, LANES), lambda i: (0, 0)),
            pl.BlockSpec((SC_ROWS * 16 // LANES, LANES), lambda i: (0, 0)),
            pl.BlockSpec((8, d), lambda i: (0, 0)),
        ],
        out_specs=pl.BlockSpec((1, 1), lambda i: (0, 0)),
        out_shape=jax.ShapeDtypeStruct((1, 1), jnp.float32),
    )(cos_tc, dot_sc.reshape(SC_ROWS * 16 // LANES, LANES),
      sq_sc.reshape(SC_ROWS * 16 // LANES, LANES), img)
    return out[0, 0]


# speedup vs baseline: 2.3889x; 1.6338x over previous
import functools

import jax
import jax.numpy as jnp
from jax.experimental import pallas as pl
from jax.experimental.pallas import tpu as pltpu

MARGIN = 0.3
K_POS = 8
K_NEG = 64
EPS = 1e-8

BLOCK = 8192
LANES = 128


def _float_keys(vals, valid):
    bits = jax.lax.bitcast_convert_type(vals, jnp.uint32)
    neg = bits >= jnp.uint32(0x80000000)
    key = jnp.where(neg, ~bits, bits | jnp.uint32(0x80000000))
    return jnp.where(valid, key, jnp.uint32(0))


def _key_to_float(key):
    neg = key < jnp.uint32(0x80000000)
    bits = jnp.where(neg, ~key, key ^ jnp.uint32(0x80000000))
    return jax.lax.bitcast_convert_type(bits, jnp.float32)


def _kth_largest(vals, valid, k):
    keys = _float_keys(vals, valid)
    prefix = jnp.uint32(0)
    for b in range(30, -1, -2):
        c3 = prefix | jnp.uint32(3 << b)
        c2 = prefix | jnp.uint32(2 << b)
        c1 = prefix | jnp.uint32(1 << b)
        n3 = jnp.sum((keys >= c3).astype(jnp.int32))
        n2 = jnp.sum((keys >= c2).astype(jnp.int32))
        n1 = jnp.sum((keys >= c1).astype(jnp.int32))
        prefix = jnp.where(
            n3 >= k, c3, jnp.where(n2 >= k, c2, jnp.where(n1 >= k, c1, prefix)))
    return _key_to_float(prefix)


def _loss_kernel(img_ref, anchor_ref, out_ref, cos_scratch, *, n_rows):
    i = pl.program_id(0)
    nblocks = pl.num_programs(0)

    x = img_ref[...]
    a = anchor_ref[0:1, :]
    xb = x.astype(jnp.bfloat16)
    ab = jnp.transpose(a.astype(jnp.bfloat16))
    dot = jax.lax.dot_general(
        xb, ab, (((1,), (0,)), ((), ())),
        preferred_element_type=jnp.float32)
    ones = jnp.ones((a.shape[1], 1), jnp.bfloat16)
    sq = jax.lax.dot_general(
        xb * xb, ones, (((1,), (0,)), ((), ())),
        preferred_element_type=jnp.float32)
    na = jnp.sqrt(jnp.sum(a * a))
    rows = BLOCK // LANES
    dot_r = dot.reshape(rows, LANES)
    sq_r = sq.reshape(rows, LANES)
    denom = jnp.maximum(na * jnp.sqrt(sq_r), EPS)
    cos_scratch[pl.ds(i * rows, rows), :] = dot_r / denom

    @pl.when(i == nblocks - 1)
    def _finish():
        cosv = cos_scratch[...]
        r_idx = jax.lax.broadcasted_iota(jnp.int32, cosv.shape, 0)
        c_idx = jax.lax.broadcasted_iota(jnp.int32, cosv.shape, 1)
        pos = r_idx * LANES + c_idx

        lane = jax.lax.broadcasted_iota(jnp.int32, (1, LANES), 1)
        vp = (lane >= 1) & (lane <= 16)
        d_row = jnp.where(vp, 1.0 - cosv[0:1, :], -3.0e38)
        d_col = jnp.transpose(d_row)
        l_col = jnp.transpose(lane)
        beats = (d_row > d_col) | ((d_row == d_col) & (lane < l_col))
        rank = jnp.sum(beats.astype(jnp.int32), axis=1, keepdims=True)
        sel8 = jnp.transpose(vp) & (rank < K_POS)
        mean_p = jnp.sum(jnp.where(sel8, d_col, 0.0)) / K_POS

        valid_n = (pos >= 17) & (pos < n_rows)
        t64 = _kth_largest(cosv, valid_n, K_NEG)
        c = mean_p + MARGIN
        h = jnp.maximum(c - (1.0 - cosv), 0.0)
        gtn = valid_n & (cosv > t64)
        gn = jnp.sum(gtn.astype(jnp.int32))
        sh = jnp.sum(jnp.where(gtn, h, 0.0))
        ht = jnp.maximum(c - (1.0 - t64), 0.0)
        loss = (sh + (K_NEG - gn).astype(jnp.float32) * ht) / K_NEG
        out_ref[...] = jnp.reshape(loss, (1, 1))


def kernel(img):
    n, d = img.shape
    nblocks = pl.cdiv(n, BLOCK)
    scratch_rows = nblocks * BLOCK // LANES
    out = pl.pallas_call(
        functools.partial(_loss_kernel, n_rows=n),
        grid=(nblocks,),
        in_specs=[
            pl.BlockSpec((BLOCK, d), lambda i: (i, 0)),
            pl.BlockSpec((8, d), lambda i: (0, 0)),
        ],
        out_specs=pl.BlockSpec((1, 1), lambda i: (0, 0)),
        out_shape=jax.ShapeDtypeStruct((1, 1), jnp.float32),
        scratch_shapes=[pltpu.VMEM((scratch_rows, LANES), jnp.float32)],
    )(img, img)
    return out[0, 0]
